# baseline (speedup 1.0000x reference)
import functools

import jax
import jax.numpy as jnp
from jax import lax
from jax.experimental import pallas as pl
from jax.experimental.pallas import tpu as pltpu
from jax.experimental.pallas import tpu_sc as plsc

EMB_DIM = 32
HIST = 50
BATCH = 16384
VOCAB = 1000000
NIDX = BATCH * HIST
NCORE = 2
NSUB = 16
CPC = EMB_DIM // NCORE
PERSUB = NIDX // NSUB
HALF = PERSUB // 2


@jax.jit
def _sc_embed(table_flat, x_flat):
    mesh = plsc.VectorSubcoreMesh(core_axis_name="c", subcore_axis_name="s")

    @functools.partial(
        pl.kernel,
        out_type=jax.ShapeDtypeStruct((EMB_DIM * NIDX,), jnp.float32),
        mesh=mesh,
        scratch_types=[
            pltpu.VMEM((HALF,), jnp.int32),
            pltpu.VMEM((HALF,), jnp.int32),
            pltpu.VMEM((HALF,), jnp.float32),
            pltpu.VMEM((HALF,), jnp.float32),
            pltpu.SemaphoreType.DMA,
        ],
    )
    def embed_kernel(
        tab_hbm, xf_hbm, out_hbm, idx_a, idx_b, rows_a, rows_b, sem
    ):
        core = lax.axis_index("c")
        s = lax.axis_index("s")
        f0 = s * PERSUB

        for half, idx_v in ((0, idx_a), (1, idx_b)):
            pltpu.sync_copy(
                xf_hbm.at[pl.ds(f0 + half * HALF, HALF)], idx_v
            )

        @pl.loop(0, CPC)
        def _(cc):
            c = core * CPC + cc
            row = tab_hbm.at[pl.ds(c * VOCAB, VOCAB)]

            for half, idx_v, rows_v in ((0, idx_a, rows_a), (1, idx_b, rows_b)):
                pltpu.sync_copy(row.at[idx_v], rows_v)
                pltpu.sync_copy(
                    rows_v,
                    out_hbm.at[pl.ds(c * NIDX + f0 + half * HALF, HALF)],
                )

    return embed_kernel(table_flat, x_flat)


def kernel(x, table):
    table_flat = table.T.reshape(EMB_DIM * VOCAB)
    x_flat = x.astype(jnp.int32).T.reshape(NIDX)
    out = _sc_embed(table_flat, x_flat)
    out = out.reshape(EMB_DIM, HIST, BATCH)
    return jnp.transpose(out, (2, 1, 0))


# speedup vs baseline: 1.0604x; 1.0604x over previous
import functools

import jax
import jax.numpy as jnp
from jax import lax
from jax.experimental import pallas as pl
from jax.experimental.pallas import tpu as pltpu
from jax.experimental.pallas import tpu_sc as plsc

EMB_DIM = 32
HIST = 50
BATCH = 16384
VOCAB = 1000000
NIDX = BATCH * HIST
NCORE = 2
NSUB = 16
PERCORE = NIDX // NCORE
PERSUB = PERCORE // NSUB
CHUNK = 4000
NCHUNK = VOCAB // CHUNK


@jax.jit
def _sc_embed(table_flat, x_flat):
    mesh = plsc.VectorSubcoreMesh(core_axis_name="c", subcore_axis_name="s")

    @functools.partial(
        pl.kernel,
        out_type=jax.ShapeDtypeStruct((EMB_DIM * NIDX,), jnp.float32),
        mesh=mesh,
        scratch_types=[
            pltpu.VMEM((PERSUB,), jnp.int32),
            pltpu.VMEM((PERSUB,), jnp.float32),
            pltpu.VMEM((CHUNK,), jnp.float32),
            pltpu.VMEM_SHARED((VOCAB,), jnp.float32),
            pltpu.SemaphoreType.DMA,
        ],
    )
    def embed_kernel(tab_hbm, xf_hbm, out_hbm, idx_v, rows_v, bounce,
                     spmem, sem):
        core = lax.axis_index("c")
        s = lax.axis_index("s")
        f0 = core * PERCORE + s * PERSUB

        pltpu.sync_copy(xf_hbm.at[pl.ds(f0, PERSUB)], idx_v)

        @pl.loop(0, EMB_DIM)
        def _(c):
            for k in range(NCHUNK // NSUB + (1 if NCHUNK % NSUB else 0)):
                j = k * NSUB + s

                @pl.when(j < NCHUNK)
                def _():
                    pltpu.sync_copy(
                        tab_hbm.at[pl.ds(c * VOCAB + j * CHUNK, CHUNK)],
                        bounce,
                    )
                    pltpu.sync_copy(bounce, spmem.at[pl.ds(j * CHUNK, CHUNK)])

            plsc.subcore_barrier()

            pltpu.sync_copy(spmem.at[idx_v], rows_v)
            pltpu.sync_copy(rows_v, out_hbm.at[pl.ds(c * NIDX + f0, PERSUB)])

            plsc.subcore_barrier()

    return embed_kernel(table_flat, x_flat)


def kernel(x, table):
    table_flat = table.T.reshape(EMB_DIM * VOCAB)
    x_flat = x.astype(jnp.int32).T.reshape(NIDX)
    out = _sc_embed(table_flat, x_flat)
    out = out.reshape(EMB_DIM, HIST, BATCH)
    return jnp.transpose(out, (2, 1, 0))


# speedup vs baseline: 2.7369x; 2.5811x over previous
import functools

import jax
import jax.numpy as jnp
from jax import lax
from jax.experimental import pallas as pl
from jax.experimental.pallas import tpu as pltpu
from jax.experimental.pallas import tpu_sc as plsc

EMB_DIM = 32
HIST = 50
BATCH = 16384
VOCAB = 1000000
NIDX = BATCH * HIST
NCORE = 2
NSUB = 16
PERCORE = NIDX // NCORE
PERSUB = PERCORE // NSUB
CHUNK = 4000
NCHUNK = VOCAB // CHUNK


RCHUNK = 12800
NFULL = VOCAB // RCHUNK
VMAIN = NFULL * RCHUNK + 1536
VTAIL = VOCAB - VMAIN
NITEM = 2 * (NFULL + 1)


@jax.jit
def _sc_relayout(table_t):
    mesh = plsc.VectorSubcoreMesh(core_axis_name="c", subcore_axis_name="s")

    @functools.partial(
        pl.kernel,
        out_type=jax.ShapeDtypeStruct((EMB_DIM * VOCAB,), jnp.float32),
        mesh=mesh,
        scratch_types=[
            pltpu.VMEM((8, RCHUNK), jnp.float32),
            pltpu.VMEM((RCHUNK,), jnp.float32),
            pltpu.SemaphoreType.DMA,
        ],
    )
    def relayout_kernel(tab_hbm, flat_hbm, blk, obuf, sem):
        core = lax.axis_index("c")
        s = lax.axis_index("s")

        @pl.loop(0, NITEM // NSUB + (1 if NITEM % NSUB else 0))
        def _(k):
            i = k * NSUB + s

            @pl.when(i < NITEM)
            def _():
                tc = core * 2 + i // (NFULL + 1)
                j = i % (NFULL + 1)

                @pl.when(j < NFULL)
                def _():
                    voff = j * RCHUNK
                    pltpu.sync_copy(
                        tab_hbm.at[pl.ds(8 * tc, 8), pl.ds(voff, RCHUNK)], blk
                    )
                    for r in range(8):
                        @pl.loop(0, RCHUNK // 16)
                        def _(k):
                            obuf[pl.ds(k * 16, 16)] = blk[r, pl.ds(k * 16, 16)]

                        pltpu.sync_copy(
                            obuf,
                            flat_hbm.at[
                                pl.ds((8 * tc + r) * VOCAB + voff, RCHUNK)
                            ],
                        )

                @pl.when(j == NFULL)
                def _():
                    voff, ln = NFULL * RCHUNK, 1536
                    pltpu.sync_copy(
                        tab_hbm.at[pl.ds(8 * tc, 8), pl.ds(voff, ln)],
                        blk.at[:, pl.ds(0, ln)],
                    )
                    for r in range(8):
                        @pl.loop(0, ln // 16)
                        def _(k):
                            obuf[pl.ds(k * 16, 16)] = (
                                blk[r, pl.ds(k * 16, 16)]
                            )

                        pltpu.sync_copy(
                            obuf.at[pl.ds(0, ln)],
                            flat_hbm.at[
                                pl.ds((8 * tc + r) * VOCAB + voff, ln)
                            ],
                        )

    return relayout_kernel(table_t)


NSCHUNK = VMAIN // CHUNK + 1


@jax.jit
def _sc_embed(table_flat, aux, x_flat):
    mesh = plsc.VectorSubcoreMesh(core_axis_name="c", subcore_axis_name="s")

    @functools.partial(
        pl.kernel,
        out_type=jax.ShapeDtypeStruct((EMB_DIM * NIDX,), jnp.float32),
        mesh=mesh,
        scratch_types=[
            pltpu.VMEM((PERSUB,), jnp.int32),
            pltpu.VMEM((PERSUB,), jnp.float32),
            pltpu.VMEM((CHUNK,), jnp.float32),
            pltpu.VMEM_SHARED((VOCAB,), jnp.float32),
            pltpu.SemaphoreType.DMA,
        ],
    )
    def embed_kernel(tab_hbm, aux_hbm, xf_hbm, out_hbm, idx_v, rows_v, bounce,
                     spmem, sem):
        core = lax.axis_index("c")
        s = lax.axis_index("s")
        f0 = core * PERCORE + s * PERSUB

        pltpu.sync_copy(xf_hbm.at[pl.ds(f0, PERSUB)], idx_v)

        @pl.loop(0, EMB_DIM)
        def _(c):
            for k in range(NSCHUNK // NSUB + (1 if NSCHUNK % NSUB else 0)):
                j = k * NSUB + s

                @pl.when(j < NSCHUNK - 1)
                def _():
                    pltpu.sync_copy(
                        tab_hbm.at[pl.ds(c * VOCAB + j * CHUNK, CHUNK)],
                        bounce,
                    )
                    pltpu.sync_copy(bounce, spmem.at[pl.ds(j * CHUNK, CHUNK)])

                @pl.when(j == NSCHUNK - 1)
                def _():
                    ln = VMAIN - (NSCHUNK - 1) * CHUNK
                    pltpu.sync_copy(
                        tab_hbm.at[pl.ds(c * VOCAB + j * CHUNK, ln)],
                        bounce.at[pl.ds(0, ln)],
                    )
                    pltpu.sync_copy(
                        bounce.at[pl.ds(0, ln)],
                        spmem.at[pl.ds(j * CHUNK, ln)],
                    )

            @pl.when(s == 0)
            def _():
                pltpu.sync_copy(
                    aux_hbm.at[pl.ds(c * VTAIL, VTAIL)],
                    bounce.at[pl.ds(0, VTAIL)],
                )
                pltpu.sync_copy(
                    bounce.at[pl.ds(0, VTAIL)],
                    spmem.at[pl.ds(VMAIN, VTAIL)],
                )

            plsc.subcore_barrier()

            pltpu.sync_copy(spmem.at[idx_v], rows_v)
            pltpu.sync_copy(rows_v, out_hbm.at[pl.ds(c * NIDX + f0, PERSUB)])

            plsc.subcore_barrier()

    return embed_kernel(table_flat, aux, x_flat)


def kernel(x, table):
    table_flat = _sc_relayout(table.T)
    aux = table.T[:, VMAIN:].reshape(EMB_DIM * VTAIL)
    x_flat = x.astype(jnp.int32).T.reshape(NIDX)
    out = _sc_embed(table_flat, aux, x_flat)
    out = out.reshape(EMB_DIM, HIST, BATCH)
    return jnp.transpose(out, (2, 1, 0))


# speedup vs baseline: 3.2642x; 1.1926x over previous
import functools

import jax
import jax.numpy as jnp
from jax import lax
from jax.experimental import pallas as pl
from jax.experimental.pallas import tpu as pltpu
from jax.experimental.pallas import tpu_sc as plsc

EMB_DIM = 32
HIST = 50
BATCH = 16384
VOCAB = 1000000
NIDX = BATCH * HIST
NCORE = 2
NSUB = 16
PERCORE = NIDX // NCORE
PERSUB = PERCORE // NSUB
CHUNK = 4000
NCHUNK = VOCAB // CHUNK


RCHUNK = 12800
NFULL = VOCAB // RCHUNK
VMAIN = NFULL * RCHUNK + 1536
VTAIL = VOCAB - VMAIN
NITEM = 2 * (NFULL + 1)


@jax.jit
def _sc_relayout(table_t):
    mesh = plsc.VectorSubcoreMesh(core_axis_name="c", subcore_axis_name="s")

    @functools.partial(
        pl.kernel,
        out_type=jax.ShapeDtypeStruct((EMB_DIM * VOCAB,), jnp.float32),
        mesh=mesh,
        scratch_types=[
            pltpu.VMEM((8, RCHUNK), jnp.float32),
            pltpu.VMEM((RCHUNK,), jnp.float32),
            pltpu.VMEM((RCHUNK,), jnp.float32),
            pltpu.SemaphoreType.DMA,
        ],
    )
    def relayout_kernel(tab_hbm, flat_hbm, blk, obuf0, obuf1, sem):
        core = lax.axis_index("c")
        s = lax.axis_index("s")

        @pl.loop(0, NITEM // NSUB + (1 if NITEM % NSUB else 0))
        def _(k):
            i = k * NSUB + s

            @pl.when(i < NITEM)
            def _():
                tc = core * 2 + i // (NFULL + 1)
                j = i % (NFULL + 1)

                obufs = (obuf0, obuf1)

                @pl.when(j < NFULL)
                def _():
                    voff = j * RCHUNK
                    pltpu.sync_copy(
                        tab_hbm.at[pl.ds(8 * tc, 8), pl.ds(voff, RCHUNK)], blk
                    )
                    for r in range(8):
                        ob = obufs[r % 2]
                        if r >= 2:
                            pltpu.make_async_copy(
                                ob,
                                flat_hbm.at[pl.ds(
                                    (8 * tc + r - 2) * VOCAB + voff, RCHUNK
                                )],
                                sem,
                            ).wait()

                        @pl.loop(0, RCHUNK // 16)
                        def _(k):
                            ob[pl.ds(k * 16, 16)] = blk[r, pl.ds(k * 16, 16)]

                        pltpu.async_copy(
                            ob,
                            flat_hbm.at[
                                pl.ds((8 * tc + r) * VOCAB + voff, RCHUNK)
                            ],
                            sem,
                        )
                    for r in (6, 7):
                        pltpu.make_async_copy(
                            obufs[r % 2],
                            flat_hbm.at[
                                pl.ds((8 * tc + r) * VOCAB + voff, RCHUNK)
                            ],
                            sem,
                        ).wait()

                @pl.when(j == NFULL)
                def _():
                    voff, ln = NFULL * RCHUNK, 1536
                    pltpu.sync_copy(
                        tab_hbm.at[pl.ds(8 * tc, 8), pl.ds(voff, ln)],
                        blk.at[:, pl.ds(0, ln)],
                    )
                    for r in range(8):
                        @pl.loop(0, ln // 16)
                        def _(k):
                            obuf0[pl.ds(k * 16, 16)] = (
                                blk[r, pl.ds(k * 16, 16)]
                            )

                        pltpu.sync_copy(
                            obuf0.at[pl.ds(0, ln)],
                            flat_hbm.at[
                                pl.ds((8 * tc + r) * VOCAB + voff, ln)
                            ],
                        )

    return relayout_kernel(table_t)


PERSTAGE = VMAIN // NSUB
SCH = 5208
NSCH = PERSTAGE // SCH


@jax.jit
def _sc_embed(table_flat, aux, x_flat):
    mesh = plsc.VectorSubcoreMesh(core_axis_name="c", subcore_axis_name="s")

    @functools.partial(
        pl.kernel,
        out_type=jax.ShapeDtypeStruct((EMB_DIM * NIDX,), jnp.float32),
        mesh=mesh,
        scratch_types=[
            pltpu.VMEM((PERSUB,), jnp.int32),
            pltpu.VMEM((PERSUB,), jnp.float32),
            pltpu.VMEM((SCH,), jnp.float32),
            pltpu.VMEM((SCH,), jnp.float32),
            pltpu.VMEM_SHARED((VOCAB,), jnp.float32),
            pltpu.SemaphoreType.DMA,
            pltpu.SemaphoreType.DMA,
            pltpu.SemaphoreType.DMA,
        ],
    )
    def embed_kernel(tab_hbm, aux_hbm, xf_hbm, out_hbm, idx_v, rows_v,
                     b0, b1, spmem, semh, sems, semw):
        core = lax.axis_index("c")
        s = lax.axis_index("s")
        f0 = core * PERCORE + s * PERSUB
        v0 = s * PERSTAGE

        pltpu.sync_copy(xf_hbm.at[pl.ds(f0, PERSUB)], idx_v)

        @pl.loop(0, EMB_DIM)
        def _(c):
            bufs = (b0, b1)

            def hsrc(m):
                return tab_hbm.at[pl.ds(c * VOCAB + v0 + m * SCH, SCH)]

            def sdst(m):
                return spmem.at[pl.ds(v0 + m * SCH, SCH)]

            hc = pltpu.async_copy(hsrc(0), b0, semh)
            s_hand = [None, None]
            for m in range(NSCH):
                bcur = bufs[m % 2]
                hc.wait()
                if m + 1 < NSCH:
                    bnext = bufs[(m + 1) % 2]
                    if s_hand[(m + 1) % 2] is not None:
                        s_hand[(m + 1) % 2].wait()
                    hc = pltpu.async_copy(hsrc(m + 1), bnext, semh)
                s_hand[m % 2] = pltpu.async_copy(bcur, sdst(m), sems)
            for h in s_hand:
                h.wait()

            @pl.when(s == 0)
            def _():
                pltpu.sync_copy(
                    aux_hbm.at[pl.ds(c * VTAIL, VTAIL)],
                    b0.at[pl.ds(0, VTAIL)],
                )
                pltpu.sync_copy(
                    b0.at[pl.ds(0, VTAIL)],
                    spmem.at[pl.ds(VMAIN, VTAIL)],
                )

            plsc.subcore_barrier()

            @pl.when(c > 0)
            def _():
                pltpu.make_async_copy(
                    rows_v, out_hbm.at[pl.ds((c - 1) * NIDX + f0, PERSUB)],
                    semw,
                ).wait()

            pltpu.sync_copy(spmem.at[idx_v], rows_v)
            pltpu.async_copy(
                rows_v, out_hbm.at[pl.ds(c * NIDX + f0, PERSUB)], semw
            )

            plsc.subcore_barrier()

        pltpu.make_async_copy(
            rows_v, out_hbm.at[pl.ds((EMB_DIM - 1) * NIDX + f0, PERSUB)], semw
        ).wait()

    return embed_kernel(table_flat, aux, x_flat)


def kernel(x, table):
    table_flat = _sc_relayout(table.T)
    aux = table.T[:, VMAIN:].reshape(EMB_DIM * VTAIL)
    x_flat = x.astype(jnp.int32).T.reshape(NIDX)
    out = _sc_embed(table_flat, aux, x_flat)
    out = out.reshape(EMB_DIM, HIST, BATCH)
    return jnp.transpose(out, (2, 1, 0))


# speedup vs baseline: 3.5481x; 1.0870x over previous
import functools

import jax
import jax.numpy as jnp
from jax import lax
from jax.experimental import pallas as pl
from jax.experimental.pallas import tpu as pltpu
from jax.experimental.pallas import tpu_sc as plsc

EMB_DIM = 32
HIST = 50
BATCH = 16384
VOCAB = 1000000
NIDX = BATCH * HIST
NCORE = 2
NSUB = 16
PERCORE = NIDX // NCORE
PERSUB = PERCORE // NSUB
CHUNK = 4000
NCHUNK = VOCAB // CHUNK


RCHUNK = 12800
NFULL = VOCAB // RCHUNK
VMAIN = NFULL * RCHUNK + 1536
VTAIL = VOCAB - VMAIN
NITEM = 2 * (NFULL + 1)


@jax.jit
def _sc_relayout(table_t):
    mesh = plsc.VectorSubcoreMesh(core_axis_name="c", subcore_axis_name="s")

    @functools.partial(
        pl.kernel,
        out_type=jax.ShapeDtypeStruct((EMB_DIM * VOCAB,), jnp.float32),
        mesh=mesh,
        scratch_types=[
            pltpu.VMEM((8, RCHUNK), jnp.float32),
            pltpu.VMEM((RCHUNK,), jnp.float32),
            pltpu.VMEM((RCHUNK,), jnp.float32),
            pltpu.SemaphoreType.DMA,
        ],
    )
    def relayout_kernel(tab_hbm, flat_hbm, blk, obuf0, obuf1, sem):
        core = lax.axis_index("c")
        s = lax.axis_index("s")

        @pl.loop(0, NITEM // NSUB + (1 if NITEM % NSUB else 0))
        def _(k):
            i = k * NSUB + s

            @pl.when(i < NITEM)
            def _():
                tc = core * 2 + i // (NFULL + 1)
                j = i % (NFULL + 1)

                obufs = (obuf0, obuf1)

                @pl.when(j < NFULL)
                def _():
                    voff = j * RCHUNK
                    pltpu.sync_copy(
                        tab_hbm.at[pl.ds(8 * tc, 8), pl.ds(voff, RCHUNK)], blk
                    )
                    for r in range(8):
                        ob = obufs[r % 2]
                        if r >= 2:
                            pltpu.make_async_copy(
                                ob,
                                flat_hbm.at[pl.ds(
                                    (8 * tc + r - 2) * VOCAB + voff, RCHUNK
                                )],
                                sem,
                            ).wait()

                        @pl.loop(0, RCHUNK // 64)
                        def _(k):
                            for u in range(4):
                                o = k * 64 + u * 16
                                ob[pl.ds(o, 16)] = blk[r, pl.ds(o, 16)]

                        pltpu.async_copy(
                            ob,
                            flat_hbm.at[
                                pl.ds((8 * tc + r) * VOCAB + voff, RCHUNK)
                            ],
                            sem,
                        )
                    for r in (6, 7):
                        pltpu.make_async_copy(
                            obufs[r % 2],
                            flat_hbm.at[
                                pl.ds((8 * tc + r) * VOCAB + voff, RCHUNK)
                            ],
                            sem,
                        ).wait()

                @pl.when(j == NFULL)
                def _():
                    voff, ln = NFULL * RCHUNK, 1536
                    pltpu.sync_copy(
                        tab_hbm.at[pl.ds(8 * tc, 8), pl.ds(voff, ln)],
                        blk.at[:, pl.ds(0, ln)],
                    )
                    for r in range(8):
                        @pl.loop(0, ln // 16)
                        def _(k):
                            obuf0[pl.ds(k * 16, 16)] = (
                                blk[r, pl.ds(k * 16, 16)]
                            )

                        pltpu.sync_copy(
                            obuf0.at[pl.ds(0, ln)],
                            flat_hbm.at[
                                pl.ds((8 * tc + r) * VOCAB + voff, ln)
                            ],
                        )

    return relayout_kernel(table_t)


PERSTAGE = VMAIN // NSUB
SCH = 5208
NSCH = PERSTAGE // SCH


@jax.jit
def _sc_embed(table_flat, aux, x_flat):
    mesh = plsc.VectorSubcoreMesh(core_axis_name="c", subcore_axis_name="s")

    @functools.partial(
        pl.kernel,
        out_type=jax.ShapeDtypeStruct((HIST * EMB_DIM * BATCH,), jnp.float32),
        mesh=mesh,
        scratch_types=[
            pltpu.VMEM((PERSUB,), jnp.int32),
            pltpu.VMEM((PERSUB,), jnp.float32),
            pltpu.VMEM((SCH,), jnp.float32),
            pltpu.VMEM((SCH,), jnp.float32),
            pltpu.VMEM_SHARED((VOCAB,), jnp.float32),
            pltpu.SemaphoreType.DMA,
            pltpu.SemaphoreType.DMA,
            pltpu.SemaphoreType.DMA,
        ],
    )
    def embed_kernel(tab_hbm, aux_hbm, xf_hbm, out_hbm, idx_v, rows_v,
                     b0, b1, spmem, semh, sems, semw):
        core = lax.axis_index("c")
        s = lax.axis_index("s")
        h0 = core * (HIST // NCORE)
        bcol = s * (BATCH // NSUB)
        BB = BATCH // NSUB
        v0 = s * PERSTAGE

        for i in range(HIST // NCORE):
            pltpu.sync_copy(
                xf_hbm.at[pl.ds((h0 + i) * BATCH + bcol, BB)],
                idx_v.at[pl.ds(i * BB, BB)],
            )

        def odst(i, cc):
            return out_hbm.at[
                pl.ds(((h0 + i) * EMB_DIM + cc) * BATCH + bcol, BB)
            ]

        @pl.loop(0, EMB_DIM)
        def _(c):
            bufs = (b0, b1)

            def hsrc(m):
                return tab_hbm.at[pl.ds(c * VOCAB + v0 + m * SCH, SCH)]

            def sdst(m):
                return spmem.at[pl.ds(v0 + m * SCH, SCH)]

            hc = pltpu.async_copy(hsrc(0), b0, semh)
            s_hand = [None, None]
            for m in range(NSCH):
                bcur = bufs[m % 2]
                hc.wait()
                if m + 1 < NSCH:
                    bnext = bufs[(m + 1) % 2]
                    if s_hand[(m + 1) % 2] is not None:
                        s_hand[(m + 1) % 2].wait()
                    hc = pltpu.async_copy(hsrc(m + 1), bnext, semh)
                s_hand[m % 2] = pltpu.async_copy(bcur, sdst(m), sems)
            for h in s_hand:
                h.wait()

            @pl.when(s == 0)
            def _():
                pltpu.sync_copy(
                    aux_hbm.at[pl.ds(c * VTAIL, VTAIL)],
                    b0.at[pl.ds(0, VTAIL)],
                )
                pltpu.sync_copy(
                    b0.at[pl.ds(0, VTAIL)],
                    spmem.at[pl.ds(VMAIN, VTAIL)],
                )

            plsc.subcore_barrier()

            @pl.when(c > 0)
            def _():
                for i in range(HIST // NCORE):
                    pltpu.make_async_copy(
                        rows_v.at[pl.ds(i * BB, BB)], odst(i, c - 1), semw
                    ).wait()

            pltpu.sync_copy(spmem.at[idx_v], rows_v)
            for i in range(HIST // NCORE):
                pltpu.async_copy(
                    rows_v.at[pl.ds(i * BB, BB)], odst(i, c), semw
                )

            plsc.subcore_barrier()

        for i in range(HIST // NCORE):
            pltpu.make_async_copy(
                rows_v.at[pl.ds(i * BB, BB)], odst(i, EMB_DIM - 1), semw
            ).wait()

    return embed_kernel(table_flat, aux, x_flat)


def kernel(x, table):
    table_flat = _sc_relayout(table.T)
    aux = table.T[:, VMAIN:].reshape(EMB_DIM * VTAIL)
    x_t = x.astype(jnp.int32).T.reshape(NIDX)
    out = _sc_embed(table_flat, aux, x_t)
    out = out.reshape(HIST, EMB_DIM, BATCH)
    return jnp.transpose(out, (2, 0, 1))
